# baseline (speedup 1.0000x reference)
import functools

import jax
import jax.numpy as jnp
from jax import lax
from jax.experimental import pallas as pl
from jax.experimental.pallas import tpu as pltpu
from jax.experimental.pallas import tpu_sc as plsc


def _make_gather(B, D, C):
    info = plsc.get_sparse_core_info()
    nc, ns = info.num_cores, info.num_subcores
    nw = nc * ns
    n_per_w = B // nw
    n_chunks = n_per_w // C
    mesh = plsc.VectorSubcoreMesh(core_axis_name="c", subcore_axis_name="s")

    @functools.partial(
        pl.kernel,
        out_type=jax.ShapeDtypeStruct((B, 2 * D), jnp.float32),
        mesh=mesh,
        scratch_types=[
            pltpu.VMEM((C,), jnp.int32),
            pltpu.VMEM((C,), jnp.int32),
            pltpu.VMEM((C, 2 * D), jnp.float32),
            pltpu.VMEM((C, 2 * D), jnp.float32),
            pltpu.SemaphoreType.DMA,
            pltpu.SemaphoreType.DMA,
            pltpu.SemaphoreType.DMA,
        ],
        compiler_params=pltpu.CompilerParams(use_tc_tiling_on_sc=True),
    )
    def k(idx_hbm, table_hbm, out_hbm, idx_a, idx_b, rows_a, rows_b, isem, gsem, wsem):
        wid = lax.axis_index("s") * nc + lax.axis_index("c")
        base0 = wid * n_per_w
        bufs = ((idx_a, rows_a), (idx_b, rows_b))

        pltpu.sync_copy(idx_hbm.at[pl.ds(base0, C)], idx_a)
        pltpu.async_copy(table_hbm.at[idx_a], rows_a, gsem)
        pltpu.async_copy(idx_hbm.at[pl.ds(base0 + C, C)], idx_b, isem)

        @pl.loop(0, n_chunks, step=2)
        def _(j):
            for b in (0, 1):
                jj = j + b
                idx_c, rows_c = bufs[b]
                idx_n, rows_n = bufs[1 - b]
                pltpu.make_async_copy(
                    table_hbm.at[pl.ds(0, C)], rows_c, gsem
                ).wait()
                pltpu.async_copy(
                    rows_c, out_hbm.at[pl.ds(base0 + jj * C, C)], wsem
                )

                @pl.when(jj + 1 < n_chunks)
                def _():
                    pltpu.make_async_copy(
                        idx_hbm.at[pl.ds(0, C)], idx_n, isem
                    ).wait()

                    @pl.when(jj > 0)
                    def _():
                        pltpu.make_async_copy(
                            out_hbm.at[pl.ds(0, C)], rows_n, wsem
                        ).wait()

                    pltpu.async_copy(table_hbm.at[idx_n], rows_n, gsem)

                    @pl.when(jj + 2 < n_chunks)
                    def _():
                        pltpu.async_copy(
                            idx_hbm.at[pl.ds(base0 + (jj + 2) * C, C)], idx_c, isem
                        )

        pltpu.make_async_copy(out_hbm.at[pl.ds(0, C)], rows_a, wsem).wait()
        pltpu.make_async_copy(out_hbm.at[pl.ds(0, C)], rows_a, wsem).wait()

    return k


def kernel(inputs, table):
    D = table.shape[1]
    B = inputs.shape[0] * inputs.shape[1]
    idx = inputs.reshape(B).astype(jnp.int32)
    tpad = jnp.pad(table, ((0, 0), (0, D)))
    out = _make_gather(B, D, 400)(idx, tpad)
    return out[:, :D].reshape(inputs.shape + (D,)), inputs


# speedup vs baseline: 1.1680x; 1.1680x over previous
import functools

import jax
import jax.numpy as jnp
from jax import lax
from jax.experimental import pallas as pl
from jax.experimental.pallas import tpu as pltpu
from jax.experimental.pallas import tpu_sc as plsc


def _make_gather(B, D, C):
    info = plsc.get_sparse_core_info()
    nc, ns = info.num_cores, info.num_subcores
    nw = nc * ns
    n_per_w = B // nw
    n_chunks = n_per_w // C
    mesh = plsc.VectorSubcoreMesh(core_axis_name="c", subcore_axis_name="s")

    @functools.partial(
        pl.kernel,
        out_type=jax.ShapeDtypeStruct((B, 2 * D), jnp.float32),
        mesh=mesh,
        scratch_types=[
            pltpu.VMEM((C,), jnp.int32),
            pltpu.VMEM((C,), jnp.int32),
            pltpu.VMEM((C, D), jnp.float32),
            pltpu.VMEM((C, D), jnp.float32),
            pltpu.SemaphoreType.DMA,
            pltpu.SemaphoreType.DMA,
            pltpu.SemaphoreType.DMA,
        ],
        compiler_params=pltpu.CompilerParams(use_tc_tiling_on_sc=False),
    )
    def k(idx_hbm, table_hbm, out_hbm, idx_a, idx_b, rows_a, rows_b, isem, gsem, wsem):
        wid = lax.axis_index("s") * nc + lax.axis_index("c")
        base0 = wid * n_per_w
        bufs = ((idx_a, rows_a), (idx_b, rows_b))

        pltpu.sync_copy(idx_hbm.at[pl.ds(base0, C)], idx_a)
        pltpu.async_copy(table_hbm.at[idx_a], rows_a, gsem)
        pltpu.async_copy(idx_hbm.at[pl.ds(base0 + C, C)], idx_b, isem)

        @pl.loop(0, n_chunks, step=2)
        def _(j):
            for b in (0, 1):
                jj = j + b
                idx_c, rows_c = bufs[b]
                idx_n, rows_n = bufs[1 - b]
                pltpu.make_async_copy(
                    table_hbm.at[pl.ds(0, C)], rows_c, gsem
                ).wait()
                pltpu.async_copy(
                    rows_c, out_hbm.at[pl.ds(base0 + jj * C, C), pl.ds(0, D)], wsem
                )

                @pl.when(jj + 1 < n_chunks)
                def _():
                    pltpu.make_async_copy(
                        idx_hbm.at[pl.ds(0, C)], idx_n, isem
                    ).wait()

                    @pl.when(jj > 0)
                    def _():
                        pltpu.make_async_copy(
                            out_hbm.at[pl.ds(0, C), pl.ds(0, D)], rows_n, wsem
                        ).wait()

                    pltpu.async_copy(table_hbm.at[idx_n], rows_n, gsem)

                    @pl.when(jj + 2 < n_chunks)
                    def _():
                        pltpu.async_copy(
                            idx_hbm.at[pl.ds(base0 + (jj + 2) * C, C)], idx_c, isem
                        )

        pltpu.make_async_copy(out_hbm.at[pl.ds(0, C), pl.ds(0, D)], rows_a, wsem).wait()
        pltpu.make_async_copy(out_hbm.at[pl.ds(0, C), pl.ds(0, D)], rows_a, wsem).wait()

    return k


def kernel(inputs, table):
    V, D = table.shape
    B = inputs.shape[0] * inputs.shape[1]
    idx2 = inputs.reshape(B).astype(jnp.int32) * 2
    tpad2 = jnp.pad(table, ((0, 0), (0, D))).reshape(2 * V, D)
    out = _make_gather(B, D, 800)(idx2, tpad2)
    return out[:, :D].reshape(inputs.shape + (D,)), inputs
